# Initial kernel scaffold; baseline (speedup 1.0000x reference)
import jax
import jax.numpy as jnp
from jax.experimental import pallas as pl


def kernel(tokens, wte_weight, learned_embedding):
    raise NotImplementedError("write your pallas kernel here")


# speedup vs baseline: 4.6993x; 4.6993x over previous
import jax
import jax.numpy as jnp
from jax import lax
from jax.experimental import pallas as pl
from jax.experimental.pallas import tpu as pltpu
from jax.experimental.pallas import tpu_sc as plsc

D = 64
B = 4096
SEQ = 200
NT = 20
CH = 96
NC = 2
NS = 16
NW = NC * NS
BPW = B // NW


def _sc_body(idx_hbm, wte_hbm, learned_hbm, out_hbm, idx_v, stage, sem_g):
    c = lax.axis_index("c")
    s = lax.axis_index("s")
    wid = s * NC + c
    base = wid * BPW

    pltpu.sync_copy(idx_hbm.at[pl.ds(base, BPW)], idx_v)
    pltpu.sync_copy(learned_hbm, stage.at[pl.ds(0, NT)])

    def row(b, carry):
        d0 = pltpu.async_copy(
            wte_hbm.at[idx_v.at[b, 0]], stage.at[pl.ds(NT, CH)], sem_g)
        d1 = pltpu.async_copy(
            wte_hbm.at[idx_v.at[b, 1]], stage.at[pl.ds(SEQ - CH, CH)], sem_g)
        d0.wait()
        d1.wait()
        pltpu.sync_copy(stage, out_hbm.at[base + b])
        return carry

    lax.fori_loop(0, BPW, row, 0)


def kernel(tokens, wte_weight, learned_embedding):
    tokens = tokens.astype(jnp.int32)
    idx3 = jnp.stack(
        [tokens[:, NT:NT + CH], tokens[:, SEQ - CH:SEQ]], axis=1)

    mesh = plsc.VectorSubcoreMesh(core_axis_name="c", subcore_axis_name="s")
    run = pl.kernel(
        _sc_body,
        out_type=jax.ShapeDtypeStruct((B, SEQ, D), jnp.float32),
        mesh=mesh,
        scratch_types=[
            pltpu.VMEM((BPW, 2, CH), jnp.int32),
            pltpu.VMEM((SEQ, D), jnp.float32),
            pltpu.SemaphoreType.DMA,
        ],
        compiler_params=pltpu.CompilerParams(use_tc_tiling_on_sc=False),
    )
    return run(idx3, wte_weight, learned_embedding)


# speedup vs baseline: 4.9684x; 1.0573x over previous
import jax
import jax.numpy as jnp
from jax import lax
from jax.experimental import pallas as pl
from jax.experimental.pallas import tpu as pltpu
from jax.experimental.pallas import tpu_sc as plsc

D = 64
B = 4096
SEQ = 200
NT = 20
CH = 96
NC = 2
NS = 16
NW = NC * NS
BPW = B // NW


def _sc_body(idx_hbm, wte_hbm, learned_hbm, out_hbm, idx_v, stage, sem_g, sem_w):
    c = lax.axis_index("c")
    s = lax.axis_index("s")
    wid = s * NC + c
    base = wid * BPW

    pltpu.sync_copy(idx_hbm.at[pl.ds(base, BPW)], idx_v)
    pltpu.sync_copy(learned_hbm, stage.at[0, pl.ds(0, NT)])
    pltpu.sync_copy(learned_hbm, stage.at[1, pl.ds(0, NT)])

    def start_gather(b, slot):
        pltpu.async_copy(
            wte_hbm.at[idx_v.at[b, 0]], stage.at[slot, pl.ds(NT, CH)],
            sem_g.at[slot])
        pltpu.async_copy(
            wte_hbm.at[idx_v.at[b, 1]], stage.at[slot, pl.ds(SEQ - CH, CH)],
            sem_g.at[slot])

    def wait_gather(b, slot):
        pltpu.make_async_copy(
            wte_hbm.at[idx_v.at[b, 0]], stage.at[slot, pl.ds(NT, CH)],
            sem_g.at[slot]).wait()
        pltpu.make_async_copy(
            wte_hbm.at[idx_v.at[b, 1]], stage.at[slot, pl.ds(SEQ - CH, CH)],
            sem_g.at[slot]).wait()

    def start_wb(b, slot):
        pltpu.async_copy(stage.at[slot], out_hbm.at[base + b], sem_w.at[slot])

    def wait_wb(b, slot):
        pltpu.make_async_copy(
            stage.at[slot], out_hbm.at[base + b], sem_w.at[slot]).wait()

    start_gather(0, 0)
    wait_gather(0, 0)
    start_wb(0, 0)
    start_gather(1, 1)

    def row(b, carry):
        slot = b & 1
        other = 1 - slot
        wait_gather(b, slot)
        start_wb(b, slot)
        wait_wb(b - 1, other)
        start_gather(b + 1, other)
        return carry

    lax.fori_loop(1, BPW - 1, row, 0)

    wait_gather(BPW - 1, 1)
    start_wb(BPW - 1, 1)
    wait_wb(BPW - 2, 0)
    wait_wb(BPW - 1, 1)


def kernel(tokens, wte_weight, learned_embedding):
    tokens = tokens.astype(jnp.int32)
    idx3 = jnp.stack(
        [tokens[:, NT:NT + CH], tokens[:, SEQ - CH:SEQ]], axis=1)

    mesh = plsc.VectorSubcoreMesh(core_axis_name="c", subcore_axis_name="s")
    run = pl.kernel(
        _sc_body,
        out_type=jax.ShapeDtypeStruct((B, SEQ, D), jnp.float32),
        mesh=mesh,
        scratch_types=[
            pltpu.VMEM((BPW, 2, CH), jnp.int32),
            pltpu.VMEM((2, SEQ, D), jnp.float32),
            pltpu.SemaphoreType.DMA((2,)),
            pltpu.SemaphoreType.DMA((2,)),
        ],
        compiler_params=pltpu.CompilerParams(use_tc_tiling_on_sc=False),
    )
    return run(idx3, wte_weight, learned_embedding)
